# baseline (speedup 1.0000x reference)
import functools

import jax
import jax.numpy as jnp
from jax import lax
from jax.experimental import pallas as pl
from jax.experimental.pallas import tpu as pltpu
from jax.experimental.pallas import tpu_sc as plsc

N = 320000
D = 128
S = 10000
H1 = 32

CHUNK = 128
NCH_TOTAL = N // CHUNK
NUM_SC = 2
NTILES = 16
NCH_SC = NCH_TOTAL // NUM_SC
BASE = NCH_SC // NTILES
EXTRA = NCH_SC - BASE * NTILES
SEG_PER_TILE = 624
ZROWS = 16

MM_BLK = 2560


def _tc_project(h, W1):
    def body(h_ref, w1_ref, g_ref):
        g_ref[...] = lax.dot_general(
            h_ref[...], w1_ref[...], (((1,), (0,)), ((), ())),
            precision=lax.Precision.HIGHEST,
            preferred_element_type=jnp.float32)

    return pl.pallas_call(
        body,
        grid=(N // MM_BLK,),
        in_specs=[
            pl.BlockSpec((MM_BLK, D), lambda i: (i, 0)),
            pl.BlockSpec((D, H1), lambda i: (0, 0)),
        ],
        out_specs=pl.BlockSpec((MM_BLK, H1), lambda i: (i, 0)),
        out_shape=jax.ShapeDtypeStruct((N, H1), jnp.float32),
    )(h, W1)


def _sc_segment_sum(g, batch_i32):
    mesh = plsc.VectorSubcoreMesh(core_axis_name="c", subcore_axis_name="s")

    @functools.partial(
        pl.kernel,
        out_type=jax.ShapeDtypeStruct((NUM_SC * S, H1), jnp.float32),
        mesh=mesh,
        scratch_types=[
            pltpu.VMEM((CHUNK, H1), jnp.float32),
            pltpu.VMEM((CHUNK, H1), jnp.float32),
            pltpu.VMEM((CHUNK,), jnp.int32),
            pltpu.VMEM((CHUNK,), jnp.int32),
            pltpu.VMEM((ZROWS, H1), jnp.float32),
            pltpu.VMEM_SHARED((S, H1), jnp.float32),
            pltpu.SemaphoreType.DMA,
            pltpu.SemaphoreType.DMA,
            pltpu.SemaphoreType.DMA,
            pltpu.SemaphoreType.DMA,
        ],
    )
    def seg_sum(g_hbm, b_hbm, out_hbm, gA, gB, iA, iB, zb, acc,
                sAh, sBh, sAi, sBi):
        c = lax.axis_index("c")
        s = lax.axis_index("s")
        nch = BASE + jnp.where(s < EXTRA, 1, 0)
        chunk0 = c * NCH_SC + s * BASE + jnp.minimum(s, EXTRA)

        z16 = jnp.zeros((16,), jnp.float32)

        @pl.loop(0, ZROWS)
        def _(r):
            @pl.loop(0, H1 // 16)
            def _(gi):
                zb[r, pl.ds(gi * 16, 16)] = z16

        @pl.loop(0, SEG_PER_TILE // ZROWS)
        def _(k):
            pltpu.sync_copy(
                zb, acc.at[pl.ds(s * SEG_PER_TILE + k * ZROWS, ZROWS)])

        @pl.when(s == NTILES - 1)
        def _():
            pltpu.sync_copy(zb, acc.at[pl.ds(NTILES * SEG_PER_TILE, ZROWS)])

        plsc.subcore_barrier()

        def start(gbuf, ibuf, sh, si, ci):
            row = ci * CHUNK
            pltpu.async_copy(g_hbm.at[pl.ds(row, CHUNK)], gbuf, sh)
            pltpu.async_copy(b_hbm.at[pl.ds(row, CHUNK)], ibuf, si)

        def finish_and_scatter(gbuf, ibuf, sh, si):
            pltpu.make_async_copy(g_hbm.at[pl.ds(0, CHUNK)], gbuf, sh).wait()
            pltpu.make_async_copy(b_hbm.at[pl.ds(0, CHUNK)], ibuf, si).wait()
            pltpu.sync_copy(gbuf, acc.at[ibuf], add=True)

        start(gA, iA, sAh, sAi, chunk0)
        start(gB, iB, sBh, sBi, chunk0 + 1)

        @pl.loop(0, BASE // 2)
        def _(p):
            finish_and_scatter(gA, iA, sAh, sAi)

            @pl.when(2 * p + 2 < nch)
            def _():
                start(gA, iA, sAh, sAi, chunk0 + 2 * p + 2)

            finish_and_scatter(gB, iB, sBh, sBi)

            @pl.when(2 * p + 3 < nch)
            def _():
                start(gB, iB, sBh, sBi, chunk0 + 2 * p + 3)

        @pl.when(nch > BASE)
        def _():
            finish_and_scatter(gA, iA, sAh, sAi)

        plsc.subcore_barrier()

        pltpu.sync_copy(
            acc.at[pl.ds(s * SEG_PER_TILE, SEG_PER_TILE)],
            out_hbm.at[pl.ds(c * S + s * SEG_PER_TILE, SEG_PER_TILE)])

        @pl.when(s == NTILES - 1)
        def _():
            pltpu.sync_copy(
                acc.at[pl.ds(NTILES * SEG_PER_TILE, ZROWS)],
                out_hbm.at[pl.ds(c * S + NTILES * SEG_PER_TILE, ZROWS)])

    return seg_sum(g, batch_i32)


def _ssp(x):
    return jnp.maximum(x, 0.0) + jnp.log1p(jnp.exp(-jnp.abs(x))) \
        - jnp.log(2.0).astype(jnp.float32)


def _tc_tail(partials, b1r, W2, b2r):
    BLK = 1000
    grid = S // BLK

    def body(p0_ref, p1_ref, b1_ref, w2_ref, b2_ref, o_ref):
        pooled = p0_ref[...] + p1_ref[...]
        t = _ssp(pooled + b1_ref[...])
        u = lax.dot_general(t, w2_ref[...], (((1,), (0,)), ((), ())),
                            precision=lax.Precision.HIGHEST,
                            preferred_element_type=jnp.float32)
        o_ref[...] = _ssp(u + b2_ref[...])

    return pl.pallas_call(
        body,
        grid=(grid,),
        in_specs=[
            pl.BlockSpec((BLK, H1), lambda i: (i, 0)),
            pl.BlockSpec((BLK, H1), lambda i: (i + grid, 0)),
            pl.BlockSpec((1, H1), lambda i: (0, 0)),
            pl.BlockSpec((H1, 1), lambda i: (0, 0)),
            pl.BlockSpec((1, 1), lambda i: (0, 0)),
        ],
        out_specs=pl.BlockSpec((BLK, 1), lambda i: (i, 0)),
        out_shape=jax.ShapeDtypeStruct((S, 1), jnp.float32),
    )(partials, partials, b1r, W2, b2r)


def kernel(h, batch, W1, b1, W2, b2):
    g = _tc_project(h, W1)
    partials = _sc_segment_sum(g, batch.astype(jnp.int32))
    return _tc_tail(partials, b1.reshape(1, H1), W2, b2.reshape(1, 1))


# speedup vs baseline: 1.1859x; 1.1859x over previous
import functools

import jax
import jax.numpy as jnp
from jax import lax
from jax.experimental import pallas as pl
from jax.experimental.pallas import tpu as pltpu
from jax.experimental.pallas import tpu_sc as plsc

N = 320000
D = 128
S = 10000
H1 = 32

CHUNK = 128
NCH_TOTAL = N // CHUNK
NUM_SC = 2
NTILES = 16
NCH_SC = NCH_TOTAL // NUM_SC
BASE = NCH_SC // NTILES
EXTRA = NCH_SC - BASE * NTILES
NBUF = 2
SEG_PER_TILE = 624
ZROWS = 16

MM_BLK = 8000


def _tc_project(h, W1):
    def body(h_ref, w1_ref, g_ref):
        g_ref[...] = lax.dot_general(
            h_ref[...], w1_ref[...], (((1,), (0,)), ((), ())),
            precision=lax.Precision.HIGHEST,
            preferred_element_type=jnp.float32)

    return pl.pallas_call(
        body,
        grid=(N // MM_BLK,),
        in_specs=[
            pl.BlockSpec((MM_BLK, D), lambda i: (i, 0)),
            pl.BlockSpec((D, H1), lambda i: (0, 0)),
        ],
        out_specs=pl.BlockSpec((MM_BLK, H1), lambda i: (i, 0)),
        out_shape=jax.ShapeDtypeStruct((N, H1), jnp.float32),
    )(h, W1)


def _sc_segment_sum(g, batch_i32):
    mesh = plsc.VectorSubcoreMesh(core_axis_name="c", subcore_axis_name="s")

    @functools.partial(
        pl.kernel,
        out_type=jax.ShapeDtypeStruct((NUM_SC * S, H1), jnp.float32),
        mesh=mesh,
        scratch_types=(
            [pltpu.VMEM((CHUNK, H1), jnp.float32) for _ in range(NBUF)]
            + [pltpu.VMEM((CHUNK,), jnp.int32) for _ in range(NBUF)]
            + [pltpu.VMEM((ZROWS, H1), jnp.float32),
               pltpu.VMEM_SHARED((S, H1), jnp.float32)]
            + [pltpu.SemaphoreType.DMA for _ in range(2 * NBUF)]
        ),
    )
    def seg_sum(g_hbm, b_hbm, out_hbm, *refs):
        gbufs = refs[0:NBUF]
        ibufs = refs[NBUF:2 * NBUF]
        zb = refs[2 * NBUF]
        acc = refs[2 * NBUF + 1]
        gsems = refs[2 * NBUF + 2:2 * NBUF + 2 + NBUF]
        isems = refs[2 * NBUF + 2 + NBUF:2 * NBUF + 2 + 2 * NBUF]

        c = lax.axis_index("c")
        s = lax.axis_index("s")
        nch = BASE + jnp.where(s < EXTRA, 1, 0)
        chunk0 = c * NCH_SC + s * BASE + jnp.minimum(s, EXTRA)

        z16 = jnp.zeros((16,), jnp.float32)

        @pl.loop(0, ZROWS)
        def _(r):
            @pl.loop(0, H1 // 16)
            def _(gi):
                zb[r, pl.ds(gi * 16, 16)] = z16

        @pl.loop(0, SEG_PER_TILE // ZROWS)
        def _(k):
            pltpu.sync_copy(
                zb, acc.at[pl.ds(s * SEG_PER_TILE + k * ZROWS, ZROWS)])

        @pl.when(s == NTILES - 1)
        def _():
            pltpu.sync_copy(zb, acc.at[pl.ds(NTILES * SEG_PER_TILE, ZROWS)])

        plsc.subcore_barrier()

        def start(j, ci):
            row = ci * CHUNK
            pltpu.async_copy(g_hbm.at[pl.ds(row, CHUNK)], gbufs[j], gsems[j])
            pltpu.async_copy(b_hbm.at[pl.ds(row, CHUNK)], ibufs[j], isems[j])

        def finish_and_scatter(j):
            pltpu.make_async_copy(
                g_hbm.at[pl.ds(0, CHUNK)], gbufs[j], gsems[j]).wait()
            pltpu.make_async_copy(
                b_hbm.at[pl.ds(0, CHUNK)], ibufs[j], isems[j]).wait()
            pltpu.sync_copy(gbufs[j], acc.at[ibufs[j]], add=True)

        for j in range(NBUF):
            start(j, chunk0 + j)

        @pl.loop(0, BASE // NBUF - 1)
        def _(p):
            for j in range(NBUF):
                finish_and_scatter(j)

                @pl.when(NBUF * p + j + NBUF < nch)
                def _():
                    start(j, chunk0 + NBUF * p + j + NBUF)

        P_TAIL = BASE // NBUF - 1
        for j in range(NBUF):
            finish_and_scatter(j)

            @pl.when(NBUF * P_TAIL + j + NBUF < nch)
            def _():
                start(j, chunk0 + NBUF * P_TAIL + j + NBUF)

        for j in range(BASE % NBUF):
            finish_and_scatter(j)

        @pl.when(nch > BASE)
        def _():
            finish_and_scatter(BASE % NBUF)

        plsc.subcore_barrier()

        pltpu.sync_copy(
            acc.at[pl.ds(s * SEG_PER_TILE, SEG_PER_TILE)],
            out_hbm.at[pl.ds(c * S + s * SEG_PER_TILE, SEG_PER_TILE)])

        @pl.when(s == NTILES - 1)
        def _():
            pltpu.sync_copy(
                acc.at[pl.ds(NTILES * SEG_PER_TILE, ZROWS)],
                out_hbm.at[pl.ds(c * S + NTILES * SEG_PER_TILE, ZROWS)])

    return seg_sum(g, batch_i32)


def _ssp(x):
    return jnp.maximum(x, 0.0) + jnp.log1p(jnp.exp(-jnp.abs(x))) \
        - jnp.log(2.0).astype(jnp.float32)


def _tc_tail(partials, b1r, W2, b2r):
    def body(p0_ref, p1_ref, b1_ref, w2_ref, b2_ref, o_ref):
        pooled = p0_ref[...] + p1_ref[...]
        t = _ssp(pooled + b1_ref[...])
        u = lax.dot_general(t, w2_ref[...], (((1,), (0,)), ((), ())),
                            precision=lax.Precision.HIGHEST,
                            preferred_element_type=jnp.float32)
        o_ref[...] = _ssp(u + b2_ref[...])

    return pl.pallas_call(
        body,
        grid=(1,),
        in_specs=[
            pl.BlockSpec((S, H1), lambda i: (0, 0)),
            pl.BlockSpec((S, H1), lambda i: (1, 0)),
            pl.BlockSpec((1, H1), lambda i: (0, 0)),
            pl.BlockSpec((H1, 1), lambda i: (0, 0)),
            pl.BlockSpec((1, 1), lambda i: (0, 0)),
        ],
        out_specs=pl.BlockSpec((S, 1), lambda i: (0, 0)),
        out_shape=jax.ShapeDtypeStruct((S, 1), jnp.float32),
    )(partials, partials, b1r, W2, b2r)


def kernel(h, batch, W1, b1, W2, b2):
    g = _tc_project(h, W1)
    partials = _sc_segment_sum(g, batch.astype(jnp.int32))
    return _tc_tail(partials, b1.reshape(1, H1), W2, b2.reshape(1, 1))


# speedup vs baseline: 1.8353x; 1.5476x over previous
import functools

import jax
import jax.numpy as jnp
from jax import lax
from jax.experimental import pallas as pl
from jax.experimental.pallas import tpu as pltpu
from jax.experimental.pallas import tpu_sc as plsc

N = 320000
D = 128
S = 10000
H1 = 32

CHUNK = 128
NUM_SC = 2
NTILES = 16
NW = NUM_SC * NTILES
NCH = N // CHUNK
CH_TILE = NCH // NW
EXTRA = NCH - CH_TILE * NW
ACC_ROWS = S
SEG_PER_TILE = 624


def _sc_segment_sum(h, batch_i32):
    mesh = plsc.VectorSubcoreMesh(core_axis_name="c", subcore_axis_name="s")

    @functools.partial(
        pl.kernel,
        out_type=jax.ShapeDtypeStruct((NUM_SC * S, D), jnp.float32),
        mesh=mesh,
        scratch_types=[
            pltpu.VMEM((CHUNK, D), jnp.float32),
            pltpu.VMEM((CHUNK, D), jnp.float32),
            pltpu.VMEM((2, CHUNK), jnp.int32),
            pltpu.VMEM_SHARED((ACC_ROWS, D), jnp.float32),
            pltpu.SemaphoreType.DMA,
            pltpu.SemaphoreType.DMA,
            pltpu.SemaphoreType.DMA,
            pltpu.SemaphoreType.DMA,
        ],
    )
    def seg_sum(h_hbm, b_hbm, out_hbm, hA, hB, ids2, acc,
                ldA, ldB, scA, scB):
        c = lax.axis_index("c")
        s = lax.axis_index("s")
        w = c * NTILES + s
        chunk0 = w * CH_TILE

        hbuf = (hA, hB)
        ld = (ldA, ldB)
        sc = (scA, scB)

        z16 = jnp.zeros((16,), jnp.float32)

        @pl.loop(0, CHUNK)
        def _(r):
            @pl.loop(0, D // 16)
            def _(gi):
                hA[r, pl.ds(gi * 16, 16)] = z16

        for z in range(SEG_PER_TILE // CHUNK):
            pltpu.sync_copy(
                hA, acc.at[pl.ds(s * SEG_PER_TILE + z * CHUNK, CHUNK)])
        pltpu.sync_copy(
            hA, acc.at[pl.ds(s * SEG_PER_TILE + SEG_PER_TILE - CHUNK, CHUNK)])

        @pl.when(s == NTILES - 1)
        def _():
            pltpu.sync_copy(hA, acc.at[pl.ds(ACC_ROWS - CHUNK, CHUNK)])

        plsc.subcore_barrier()

        def load(b, k):
            row = (chunk0 + k) * CHUNK
            pltpu.async_copy(h_hbm.at[pl.ds(row, CHUNK)], hbuf[b], ld[b])
            pltpu.async_copy(b_hbm.at[pl.ds(row, CHUNK)], ids2.at[b], ld[b])

        def wait_ld(b):
            pltpu.make_async_copy(
                h_hbm.at[pl.ds(0, CHUNK)], hbuf[b], ld[b]).wait()
            pltpu.make_async_copy(
                b_hbm.at[pl.ds(0, CHUNK)], ids2.at[b], ld[b]).wait()

        def fire(b):
            pltpu.async_copy(hbuf[b], acc.at[ids2.at[b]], sc[b], add=True)

        def drain(b):
            pltpu.make_async_copy(
                hbuf[b], acc.at[ids2.at[b]], sc[b]).wait()

        load(0, 0)
        load(1, 1)
        for k in range(CH_TILE):
            b = k % 2
            wait_ld(b)
            fire(b)
            if k >= 1:
                drain(1 - b)
                if k + 1 < CH_TILE:
                    load(1 - b, k + 1)
        drain((CH_TILE - 1) % 2)

        for e in range(EXTRA):
            @pl.when(w == NW - EXTRA + e)
            def _():
                row = (NCH - EXTRA + e) * CHUNK
                pltpu.async_copy(h_hbm.at[pl.ds(row, CHUNK)], hA, ldA)
                pltpu.async_copy(b_hbm.at[pl.ds(row, CHUNK)], ids2.at[0],
                                 ldA)
                pltpu.make_async_copy(
                    h_hbm.at[pl.ds(0, CHUNK)], hA, ldA).wait()
                pltpu.make_async_copy(
                    b_hbm.at[pl.ds(0, CHUNK)], ids2.at[0], ldA).wait()
                pltpu.sync_copy(hA, acc.at[ids2.at[0]], add=True)

        plsc.subcore_barrier()

        pltpu.sync_copy(
            acc.at[pl.ds(s * SEG_PER_TILE, SEG_PER_TILE)],
            out_hbm.at[pl.ds(c * S + s * SEG_PER_TILE, SEG_PER_TILE)])

        @pl.when(s == NTILES - 1)
        def _():
            pltpu.sync_copy(
                acc.at[pl.ds(NTILES * SEG_PER_TILE,
                             S - NTILES * SEG_PER_TILE)],
                out_hbm.at[pl.ds(c * S + NTILES * SEG_PER_TILE,
                                 S - NTILES * SEG_PER_TILE)])

    return seg_sum(h, batch_i32)


def _ssp(x):
    return jnp.maximum(x, 0.0) + jnp.log1p(jnp.exp(-jnp.abs(x))) \
        - jnp.log(2.0).astype(jnp.float32)


def _tc_tail(partials, W1, b1r, W2, b2r):
    def body(p0_ref, p1_ref, w1_ref, b1_ref, w2_ref, b2_ref, o_ref):
        pooled = p0_ref[...] + p1_ref[...]
        t = lax.dot_general(pooled, w1_ref[...], (((1,), (0,)), ((), ())),
                            precision=lax.Precision.HIGHEST,
                            preferred_element_type=jnp.float32)
        t = _ssp(t + b1_ref[...])
        u = lax.dot_general(t, w2_ref[...], (((1,), (0,)), ((), ())),
                            precision=lax.Precision.HIGHEST,
                            preferred_element_type=jnp.float32)
        o_ref[...] = _ssp(u + b2_ref[...])

    return pl.pallas_call(
        body,
        grid=(2,),
        in_specs=[
            pl.BlockSpec((S // 2, D), lambda i: (i, 0)),
            pl.BlockSpec((S // 2, D), lambda i: (i + 2, 0)),
            pl.BlockSpec((D, H1), lambda i: (0, 0)),
            pl.BlockSpec((1, H1), lambda i: (0, 0)),
            pl.BlockSpec((H1, 1), lambda i: (0, 0)),
            pl.BlockSpec((1, 1), lambda i: (0, 0)),
        ],
        out_specs=pl.BlockSpec((S // 2, 1), lambda i: (i, 0)),
        out_shape=jax.ShapeDtypeStruct((S, 1), jnp.float32),
    )(partials, partials, W1, b1r, W2, b2r)


def kernel(h, batch, W1, b1, W2, b2):
    partials = _sc_segment_sum(h, batch.astype(jnp.int32))
    return _tc_tail(partials, W1, b1.reshape(1, H1), W2, b2.reshape(1, 1))
